# baseline (speedup 1.0000x reference)
import functools

import jax
import jax.numpy as jnp
from jax import lax
from jax.experimental import pallas as pl
from jax.experimental.pallas import tpu as pltpu
from jax.experimental.pallas import tpu_sc as plsc

B = 4096 * 200
D = 64
NC, NS = 2, 16
NW = NC * NS
G = 128
GROUPS = B // (G * NW)
CG = 8
N_CHUNKS = GROUPS // CG


def _emb_kernel(idx_hbm, table_hbm, out_hbm, idx_v, rows_v, sem):
    wid = lax.axis_index("s") * NC + lax.axis_index("c")
    base_g = wid * GROUPS

    def body(i, carry):
        off_g = base_g + i * CG
        pltpu.sync_copy(idx_hbm.at[pl.ds(off_g, CG)], idx_v)
        copies = []
        for j in range(CG):
            copies.append(
                pltpu.async_copy(
                    table_hbm.at[idx_v.at[j]],
                    rows_v.at[pl.ds(j * G, G)],
                    sem,
                )
            )
        for c in copies:
            c.wait()
        pltpu.sync_copy(rows_v, out_hbm.at[pl.ds(off_g * G, CG * G)])
        return carry

    lax.fori_loop(0, N_CHUNKS, body, 0)


def kernel(text, table):
    idx = text.reshape(B // G, G).astype(jnp.int32)
    mesh = plsc.VectorSubcoreMesh(core_axis_name="c", subcore_axis_name="s")

    run = functools.partial(
        pl.kernel,
        out_type=jax.ShapeDtypeStruct((B, D), jnp.float32),
        mesh=mesh,
        scratch_types=[
            pltpu.VMEM((CG, G), jnp.int32),
            pltpu.VMEM((CG * G, D), jnp.float32),
            pltpu.SemaphoreType.DMA,
        ],
        compiler_params=pltpu.CompilerParams(use_tc_tiling_on_sc=False),
    )(_emb_kernel)

    out = run(idx, table)
    return out.reshape(text.shape[0], text.shape[1], D)


# speedup vs baseline: 1.0080x; 1.0080x over previous
import functools

import jax
import jax.numpy as jnp
from jax import lax
from jax.experimental import pallas as pl
from jax.experimental.pallas import tpu as pltpu
from jax.experimental.pallas import tpu_sc as plsc

B = 4096 * 200
D = 64
NC, NS = 2, 16
NW = NC * NS
G = 128
GROUPS = B // (G * NW)
CG = 5
N_CHUNKS = GROUPS // CG


def _emb_kernel(idx_hbm, table_hbm, out_hbm, idx_v, rows_v,
                gsem0, gsem1, ssem0, ssem1):
    wid = lax.axis_index("s") * NC + lax.axis_index("c")
    base_g = wid * GROUPS
    pltpu.sync_copy(idx_hbm.at[pl.ds(base_g, GROUPS)], idx_v)

    def g_copies(c, b, sem):
        return [
            pltpu.make_async_copy(
                table_hbm.at[idx_v.at[c * CG + j]],
                rows_v.at[b, pl.ds(j * G, G)],
                sem,
            )
            for j in range(CG)
        ]

    def s_copy(c, b, sem):
        return pltpu.make_async_copy(
            rows_v.at[b],
            out_hbm.at[pl.ds((base_g + c * CG) * G, CG * G)],
            sem,
        )

    def fire_gather(c, b, sem):
        for cp in g_copies(c, b, sem):
            cp.start()

    def wait_gather(c, b, sem):
        for cp in g_copies(c, b, sem):
            cp.wait()

    fire_gather(0, 0, gsem0)
    fire_gather(1, 1, gsem1)

    def body(i, carry):
        c0 = 2 * i
        wait_gather(c0, 0, gsem0)
        s_copy(c0, 0, ssem0).start()
        wait_gather(c0 + 1, 1, gsem1)
        s_copy(c0 + 1, 1, ssem1).start()
        s_copy(c0, 0, ssem0).wait()
        fire_gather(c0 + 2, 0, gsem0)
        s_copy(c0 + 1, 1, ssem1).wait()
        fire_gather(c0 + 3, 1, gsem1)
        return carry

    lax.fori_loop(0, N_CHUNKS // 2 - 1, body, 0)

    cl = N_CHUNKS - 2
    wait_gather(cl, 0, gsem0)
    s_copy(cl, 0, ssem0).start()
    wait_gather(cl + 1, 1, gsem1)
    s_copy(cl + 1, 1, ssem1).start()
    s_copy(cl, 0, ssem0).wait()
    s_copy(cl + 1, 1, ssem1).wait()


def kernel(text, table):
    idx = text.reshape(B // G, G).astype(jnp.int32)
    mesh = plsc.VectorSubcoreMesh(core_axis_name="c", subcore_axis_name="s")

    run = functools.partial(
        pl.kernel,
        out_type=jax.ShapeDtypeStruct((B, D), jnp.float32),
        mesh=mesh,
        scratch_types=[
            pltpu.VMEM((GROUPS, G), jnp.int32),
            pltpu.VMEM((2, CG * G, D), jnp.float32),
            pltpu.SemaphoreType.DMA,
            pltpu.SemaphoreType.DMA,
            pltpu.SemaphoreType.DMA,
            pltpu.SemaphoreType.DMA,
        ],
        compiler_params=pltpu.CompilerParams(use_tc_tiling_on_sc=False),
    )(_emb_kernel)

    out = run(idx, table)
    return out.reshape(text.shape[0], text.shape[1], D)


# speedup vs baseline: 1.0432x; 1.0349x over previous
import functools

import jax
import jax.numpy as jnp
from jax import lax
from jax.experimental import pallas as pl
from jax.experimental.pallas import tpu as pltpu
from jax.experimental.pallas import tpu_sc as plsc

BATCH = 4096
T = 200
D = 64
NC, NS = 2, 16
NW = NC * NS
G = 128
CG = 5
N_CHUNKS = T // CG


def _emb_kernel(idx_hbm, table_hbm, out_hbm, idx_v, rows_v,
                gsem0, gsem1, ssem0, ssem1):
    wid = lax.axis_index("s") * NC + lax.axis_index("c")
    b0 = wid * G

    def load_idx(c, b):
        pltpu.sync_copy(idx_hbm.at[pl.ds(c * CG, CG), pl.ds(b0, G)],
                        idx_v.at[b])

    def g_copies(b, sem):
        return [
            pltpu.make_async_copy(
                table_hbm.at[idx_v.at[b, j]],
                rows_v.at[b, j],
                sem,
            )
            for j in range(CG)
        ]

    def s_copy(c, b, sem):
        return pltpu.make_async_copy(
            rows_v.at[b],
            out_hbm.at[pl.ds(c * CG, CG), pl.ds(b0, G), :],
            sem,
        )

    def fire_gather(c, b, sem):
        load_idx(c, b)
        for cp in g_copies(b, sem):
            cp.start()

    def wait_gather(b, sem):
        for cp in g_copies(b, sem):
            cp.wait()

    fire_gather(0, 0, gsem0)
    fire_gather(1, 1, gsem1)

    def body(i, carry):
        c0 = 2 * i
        wait_gather(0, gsem0)
        s_copy(c0, 0, ssem0).start()
        wait_gather(1, gsem1)
        s_copy(c0 + 1, 1, ssem1).start()
        s_copy(c0, 0, ssem0).wait()
        fire_gather(c0 + 2, 0, gsem0)
        s_copy(c0 + 1, 1, ssem1).wait()
        fire_gather(c0 + 3, 1, gsem1)
        return carry

    lax.fori_loop(0, N_CHUNKS // 2 - 1, body, 0)

    cl = N_CHUNKS - 2
    wait_gather(0, gsem0)
    s_copy(cl, 0, ssem0).start()
    wait_gather(1, gsem1)
    s_copy(cl + 1, 1, ssem1).start()
    s_copy(cl, 0, ssem0).wait()
    s_copy(cl + 1, 1, ssem1).wait()


def kernel(text, table):
    idx = text.T
    mesh = plsc.VectorSubcoreMesh(core_axis_name="c", subcore_axis_name="s")

    run = functools.partial(
        pl.kernel,
        out_type=jax.ShapeDtypeStruct((T, BATCH, D), jnp.float32),
        mesh=mesh,
        scratch_types=[
            pltpu.VMEM((2, CG, G), jnp.int32),
            pltpu.VMEM((2, CG, G, D), jnp.float32),
            pltpu.SemaphoreType.DMA,
            pltpu.SemaphoreType.DMA,
            pltpu.SemaphoreType.DMA,
            pltpu.SemaphoreType.DMA,
        ],
        compiler_params=pltpu.CompilerParams(use_tc_tiling_on_sc=False),
    )(_emb_kernel)

    out = run(idx, table)
    return out.transpose(1, 0, 2)
